# baseline (speedup 1.0000x reference)
import functools

import jax
import jax.numpy as jnp
from jax import lax
from jax.experimental import pallas as pl
from jax.experimental.pallas import tpu as pltpu
from jax.experimental.pallas import tpu_sc as plsc

_BATCH = 4
_NUM_TOKENS = 8192
_EMBED = 1024
_NW = 32
_TOK_PER_W = _NUM_TOKENS // _NW
_C = 4
_STEPS = _TOK_PER_W // _C
_RING = 6
_LOOK = 4
_LOOP_STEPS = 66
_LANES = 16


def _sc_body(tok_hbm, tab_hbm, out_hbm, *scratch):
    tok_bufs = scratch[0:_RING]
    tab_bufs = scratch[_RING:2 * _RING]
    ltok_sems = scratch[2 * _RING:3 * _RING]
    ltab_sems = scratch[3 * _RING:4 * _RING]
    out_sems = scratch[4 * _RING:5 * _RING]

    wid = lax.axis_index("s") * 2 + lax.axis_index("c")
    row0 = wid * _TOK_PER_W

    def issue_load(c, q):
        r = row0 + c * _C
        pltpu.async_copy(tok_hbm.at[:, pl.ds(r, _C), :], tok_bufs[q],
                         ltok_sems[q])
        pltpu.async_copy(tab_hbm.at[pl.ds(r, _C), :], tab_bufs[q],
                         ltab_sems[q])

    def wait_load(c, s):
        r = row0 + c * _C
        pltpu.make_async_copy(tok_hbm.at[:, pl.ds(r, _C), :], tok_bufs[s],
                              ltok_sems[s]).wait()
        pltpu.make_async_copy(tab_hbm.at[pl.ds(r, _C), :], tab_bufs[s],
                              ltab_sems[s]).wait()

    def issue_store(c, s):
        r = row0 + c * _C
        pltpu.async_copy(tok_bufs[s], out_hbm.at[:, pl.ds(r, _C), :],
                         out_sems[s])

    def wait_store(c, q):
        r = row0 + c * _C
        pltpu.make_async_copy(tok_bufs[q], out_hbm.at[:, pl.ds(r, _C), :],
                              out_sems[q]).wait()

    def compute(s):
        for r in range(_C):
            def cbody(ii, carry, r=r):
                for u in range(4):
                    c0 = (ii * 4 + u) * _LANES
                    vt = tab_bufs[s][r, pl.ds(c0, _LANES)]
                    for b in range(_BATCH):
                        plsc.addupdate(
                            tok_bufs[s].at[b, r, pl.ds(c0, _LANES)], vt)
                return carry
            lax.fori_loop(0, _EMBED // _LANES // 4, cbody, 0)

    for p in range(_LOOK):
        issue_load(p, p)

    def mbody(m, carry):
        for j in range(_RING):
            c = m * _RING + j
            s = j
            q = (j + _LOOK) % _RING
            @pl.when(c >= _RING - _LOOK)
            def _():
                wait_store(c + _LOOK - _RING, q)
            @pl.when(c + _LOOK < _STEPS)
            def _():
                issue_load(c + _LOOK, q)
            @pl.when(c < _STEPS)
            def _():
                wait_load(c, s)
                compute(s)
                issue_store(c, s)
        return carry

    lax.fori_loop(0, _LOOP_STEPS // _RING, mbody, 0)

    for t in range(_LOOP_STEPS - (_RING - _LOOK), _STEPS):
        wait_store(t, t % _RING)


@functools.lru_cache(maxsize=1)
def _make_sc_add():
    return functools.partial(
        pl.kernel,
        mesh=plsc.VectorSubcoreMesh(core_axis_name="c", subcore_axis_name="s"),
        out_type=jax.ShapeDtypeStruct((_BATCH, _NUM_TOKENS, _EMBED),
                                      jnp.float32),
        scratch_types=(
            [pltpu.VMEM((_BATCH, _C, _EMBED), jnp.float32)
             for _ in range(_RING)]
            + [pltpu.VMEM((_C, _EMBED), jnp.float32) for _ in range(_RING)]
            + [pltpu.SemaphoreType.DMA for _ in range(3 * _RING)]
        ),
    )(_sc_body)


def kernel(encoded_tokens, pos_table):
    return _make_sc_add()(encoded_tokens, pos_table)


# speedup vs baseline: 1.0171x; 1.0171x over previous
import functools

import jax
import jax.numpy as jnp
from jax import lax
from jax.experimental import pallas as pl
from jax.experimental.pallas import tpu as pltpu
from jax.experimental.pallas import tpu_sc as plsc

_BATCH = 4
_NUM_TOKENS = 8192
_EMBED = 1024
_NW = 32
_TOK_PER_W = _NUM_TOKENS // _NW
_C = 4
_STEPS = _TOK_PER_W // _C
_RING = 6
_LOOK = 3
_LOOP_STEPS = 66
_LANES = 16


def _sc_body(tok_hbm, tab_hbm, out_hbm, *scratch):
    tok_bufs = scratch[0:_RING]
    tab_bufs = scratch[_RING:2 * _RING]
    ltok_sems = scratch[2 * _RING:3 * _RING]
    ltab_sems = scratch[3 * _RING:4 * _RING]
    out_sems = scratch[4 * _RING:5 * _RING]

    wid = lax.axis_index("s") * 2 + lax.axis_index("c")
    row0 = wid * _TOK_PER_W

    def issue_load(c, q):
        r = row0 + c * _C
        pltpu.async_copy(tok_hbm.at[:, pl.ds(r, _C), :], tok_bufs[q],
                         ltok_sems[q])
        pltpu.async_copy(tab_hbm.at[pl.ds(r, _C), :], tab_bufs[q],
                         ltab_sems[q])

    def wait_load(c, s):
        r = row0 + c * _C
        pltpu.make_async_copy(tok_hbm.at[:, pl.ds(r, _C), :], tok_bufs[s],
                              ltok_sems[s]).wait()
        pltpu.make_async_copy(tab_hbm.at[pl.ds(r, _C), :], tab_bufs[s],
                              ltab_sems[s]).wait()

    def issue_store(c, s):
        r = row0 + c * _C
        pltpu.async_copy(tok_bufs[s], out_hbm.at[:, pl.ds(r, _C), :],
                         out_sems[s])

    def wait_store(c, q):
        r = row0 + c * _C
        pltpu.make_async_copy(tok_bufs[q], out_hbm.at[:, pl.ds(r, _C), :],
                              out_sems[q]).wait()

    def compute(s):
        for r in range(_C):
            def cbody(ii, carry, r=r):
                for u in range(8):
                    c0 = (ii * 8 + u) * _LANES
                    vt = tab_bufs[s][r, pl.ds(c0, _LANES)]
                    for b in range(_BATCH):
                        plsc.addupdate(
                            tok_bufs[s].at[b, r, pl.ds(c0, _LANES)], vt)
                return carry
            lax.fori_loop(0, _EMBED // _LANES // 8, cbody, 0)

    for p in range(_LOOK):
        issue_load(p, p)

    def mbody(m, carry):
        for j in range(_RING):
            c = m * _RING + j
            s = j
            q = (j + _LOOK) % _RING
            @pl.when(c >= _RING - _LOOK)
            def _():
                wait_store(c + _LOOK - _RING, q)
            @pl.when(c + _LOOK < _STEPS)
            def _():
                issue_load(c + _LOOK, q)
            @pl.when(c < _STEPS)
            def _():
                wait_load(c, s)
                compute(s)
                issue_store(c, s)
        return carry

    lax.fori_loop(0, _LOOP_STEPS // _RING, mbody, 0)

    for t in range(_LOOP_STEPS - (_RING - _LOOK), _STEPS):
        wait_store(t, t % _RING)


@functools.lru_cache(maxsize=1)
def _make_sc_add():
    return functools.partial(
        pl.kernel,
        mesh=plsc.VectorSubcoreMesh(core_axis_name="c", subcore_axis_name="s"),
        out_type=jax.ShapeDtypeStruct((_BATCH, _NUM_TOKENS, _EMBED),
                                      jnp.float32),
        scratch_types=(
            [pltpu.VMEM((_BATCH, _C, _EMBED), jnp.float32)
             for _ in range(_RING)]
            + [pltpu.VMEM((_C, _EMBED), jnp.float32) for _ in range(_RING)]
            + [pltpu.SemaphoreType.DMA for _ in range(3 * _RING)]
        ),
    )(_sc_body)


def kernel(encoded_tokens, pos_table):
    return _make_sc_add()(encoded_tokens, pos_table)
